# baseline (speedup 1.0000x reference)
import functools

import jax
import jax.numpy as jnp
from jax import lax
from jax.experimental import pallas as pl
from jax.experimental.pallas import tpu as pltpu
from jax.experimental.pallas import tpu_sc as plsc

N = 100000
F = 64
K = 9
C = 80
G = N // C
NW = 32
IPS = 120
NSUB = C * K // IPS
T2MAX = (G + 2 * NW - 1) // (2 * NW)


_GDN = lax.GatherDimensionNumbers(offset_dims=(), collapsed_slice_dims=(0,),
                                  start_index_map=(0,))


def _permute(x, idx2d):
    return lax.gather(x, idx2d, _GDN, (1,),
                      mode=lax.GatherScatterMode.PROMISE_IN_BOUNDS)


def _lane_sum(x, perms):
    for p in perms:
        x = x + _permute(x, p)
    return x


def _newton_sqrt(x):
    i = lax.bitcast_convert_type(x, jnp.int32)
    y = lax.bitcast_convert_type(jnp.int32(0x5F3759DF) - (i >> 1), jnp.float32)
    hx = 0.5 * x
    for _ in range(2):
        y = y * (1.5 - hx * y * y)
    return x * y


def _sc_body(lat_hbm, hs_hbm, nbr_hbm, par_hbm, bias_hbm,
             aflow_hbm, w_hbm,
             idx0, idx1, rows0, rows1, lat0, lat1, out0, out1, wv0, wv1,
             par_v, bias_v,
             sg0, sg1, sl0, sl1, so0, so1):
    idx_v = (idx0, idx1)
    rows_v = (rows0, rows1)
    lat_v = (lat0, lat1)
    out_v = (out0, out1)
    w_v = (wv0, wv1)
    sem_g = (sg0, sg1)
    sem_l = (sl0, sl1)
    sem_o = (so0, so1)

    wid = lax.axis_index("s") * 2 + lax.axis_index("c")
    pltpu.sync_copy(par_hbm, par_v)
    pltpu.sync_copy(bias_hbm, bias_v)
    alpha_v = par_v[0]
    beta_v = par_v[1]
    biasj = [bias_v[pl.ds(16 * j, 16)] for j in range(4)]
    iota = lax.iota(jnp.int32, 16)
    mask9 = iota < K
    perms = [jnp.bitwise_xor(iota, sh)[:, None] for sh in (8, 4, 2, 1)]
    kfull = [jnp.full((16, 1), k, jnp.int32) for k in range(K)]
    kmask = [iota == k for k in range(K)]

    def issue_in(b, g):
        pltpu.sync_copy(nbr_hbm.at[g], idx_v[b])
        for jj in range(NSUB):
            pltpu.async_copy(hs_hbm.at[idx_v[b].at[jj]],
                             rows_v[b].at[pl.ds(jj * IPS, IPS)], sem_g[b])
        pltpu.async_copy(lat_hbm.at[g], lat_v[b], sem_l[b])

    def wait_in(b):
        for jj in range(NSUB):
            pltpu.make_async_copy(hs_hbm.at[idx_v[b].at[jj]],
                                  rows_v[b].at[pl.ds(jj * IPS, IPS)],
                                  sem_g[b]).wait()
        pltpu.make_async_copy(lat_hbm.at[0], lat_v[b], sem_l[b]).wait()

    def issue_out(b, g):
        pltpu.async_copy(out_v[b], aflow_hbm.at[g], sem_o[b])
        pltpu.async_copy(w_v[b], w_hbm.at[g], sem_o[b])

    def wait_out(b):
        pltpu.make_async_copy(out_v[b], aflow_hbm.at[0], sem_o[b]).wait()
        pltpu.make_async_copy(w_v[b], w_hbm.at[0], sem_o[b]).wait()

    def compute(b):
        rows = rows_v[b]
        lat = lat_v[b]
        out = out_v[b]
        wsc = w_v[b]

        def vbody(v, vc):
            latj = [lat[v, pl.ds(16 * j, 16)] for j in range(4)]
            ssvec = jnp.zeros((16,), jnp.float32)
            for k in range(K):
                acc = None
                for j in range(4):
                    d = rows[v * K + k, pl.ds(16 * j, 16)] - latj[j]
                    acc = d * d if acc is None else acc + d * d
                acc = _lane_sum(acc, perms)
                ssvec = jnp.where(kmask[k], acc, ssvec)
            svec = _newton_sqrt(ssvec)
            dsum = _lane_sum(svec, perms)
            av = alpha_v * dsum
            bv = beta_v / dsum
            wv = (av - jnp.minimum(svec, av)) * bv
            plsc.store_scatter(wsc, [v * K + iota], wv, mask=mask9)
            ojs = list(biasj)
            for k in range(K):
                wk = _permute(wv, kfull[k])
                for j in range(4):
                    ojs[j] = ojs[j] + rows[v * K + k, pl.ds(16 * j, 16)] * wk
            for j in range(4):
                out[v, pl.ds(16 * j, 16)] = ojs[j]
            return vc

        lax.fori_loop(0, C, vbody, 0, unroll=2)

    issue_in(0, wid)

    def outer(t2, carry):
        for b in (0, 1):
            t = t2 * 2 + b
            g = wid + t * NW
            gn = g + NW

            @pl.when(gn < G)
            def _prefetch(b=b, gn=gn):
                issue_in(1 - b, gn)

            @pl.when(g < G)
            def _work(b=b, g=g, t2=t2):
                wait_in(b)

                @pl.when(t2 >= 1)
                def _drain(b=b):
                    wait_out(b)

                compute(b)
                issue_out(b, g)

        return carry

    lax.fori_loop(0, T2MAX, outer, 0)
    wait_out(0)
    wait_out(1)


@jax.jit
def _run(lat3, hidden_state, nbr3, params, bias):
    mesh = plsc.VectorSubcoreMesh(core_axis_name="c", subcore_axis_name="s")
    f = functools.partial(
        pl.kernel,
        mesh=mesh,
        out_type=[
            jax.ShapeDtypeStruct((G, C, F), jnp.float32),
            jax.ShapeDtypeStruct((G, C * K), jnp.float32),
        ],
        scratch_types=[
            pltpu.VMEM((NSUB, IPS), jnp.int32),
            pltpu.VMEM((NSUB, IPS), jnp.int32),
            pltpu.VMEM((C * K, F), jnp.float32),
            pltpu.VMEM((C * K, F), jnp.float32),
            pltpu.VMEM((C, F), jnp.float32),
            pltpu.VMEM((C, F), jnp.float32),
            pltpu.VMEM((C, F), jnp.float32),
            pltpu.VMEM((C, F), jnp.float32),
            pltpu.VMEM((C * K,), jnp.float32),
            pltpu.VMEM((C * K,), jnp.float32),
            pltpu.VMEM((2, 16), jnp.float32),
            pltpu.VMEM((F,), jnp.float32),
            pltpu.SemaphoreType.DMA,
            pltpu.SemaphoreType.DMA,
            pltpu.SemaphoreType.DMA,
            pltpu.SemaphoreType.DMA,
            pltpu.SemaphoreType.DMA,
            pltpu.SemaphoreType.DMA,
        ],
        compiler_params=pltpu.CompilerParams(needs_layout_passes=False,
                                             use_tc_tiling_on_sc=False),
    )(_sc_body)
    return f(lat3, hidden_state, nbr3, params, bias)


def kernel(lattice_values, hidden_state, neighbor_idx, alpha, beta, bias):
    nbr3 = neighbor_idx.reshape(G, NSUB, IPS)
    lat3 = lattice_values.reshape(G, C, F)
    params = jnp.stack([jnp.full((16,), alpha, jnp.float32),
                        jnp.full((16,), beta, jnp.float32)])
    aflow3, w2 = _run(lat3, hidden_state, nbr3, params, bias)
    return aflow3.reshape(N, F), w2.reshape(N, K), neighbor_idx


# speedup vs baseline: 1.0243x; 1.0243x over previous
import functools

import jax
import jax.numpy as jnp
from jax import lax
from jax.experimental import pallas as pl
from jax.experimental.pallas import tpu as pltpu
from jax.experimental.pallas import tpu_sc as plsc

N = 100000
F = 64
K = 9
C = 80
G = N // C
NW = 32
IPS = 120
NSUB = C * K // IPS
T2MAX = (G + 2 * NW - 1) // (2 * NW)


_GDN = lax.GatherDimensionNumbers(offset_dims=(), collapsed_slice_dims=(0,),
                                  start_index_map=(0,))


def _permute(x, idx2d):
    return lax.gather(x, idx2d, _GDN, (1,),
                      mode=lax.GatherScatterMode.PROMISE_IN_BOUNDS)


def _lane_sum(x, perms):
    for p in perms:
        x = x + _permute(x, p)
    return x


def _newton_sqrt(x):
    i = lax.bitcast_convert_type(x, jnp.int32)
    y = lax.bitcast_convert_type(jnp.int32(0x5F3759DF) - (i >> 1), jnp.float32)
    hx = 0.5 * x
    for _ in range(2):
        y = y * (1.5 - hx * y * y)
    return x * y


def _sc_body(lat_hbm, hs_hbm, nbr_hbm, par_hbm, bias_hbm,
             aflow_hbm, w_hbm,
             idx0, idx1, rows0, rows1, lat0, lat1, out0, out1, wv0, wv1,
             par_v, bias_v,
             sg0, sg1, sl0, sl1, so0, so1):
    idx_v = (idx0, idx1)
    rows_v = (rows0, rows1)
    lat_v = (lat0, lat1)
    out_v = (out0, out1)
    w_v = (wv0, wv1)
    sem_g = (sg0, sg1)
    sem_l = (sl0, sl1)
    sem_o = (so0, so1)

    wid = lax.axis_index("s") * 2 + lax.axis_index("c")
    pltpu.sync_copy(par_hbm, par_v)
    pltpu.sync_copy(bias_hbm, bias_v)
    alpha_v = par_v[0]
    beta_v = par_v[1]
    biasj = [bias_v[pl.ds(16 * j, 16)] for j in range(4)]
    iota = lax.iota(jnp.int32, 16)
    mask9 = iota < K
    perms = [jnp.bitwise_xor(iota, sh)[:, None] for sh in (8, 4, 2, 1)]
    kfull = [jnp.full((16, 1), k, jnp.int32) for k in range(K)]
    kmask = [iota == k for k in range(K)]

    def issue_in(b, g):
        pltpu.sync_copy(nbr_hbm.at[g], idx_v[b])
        for jj in range(NSUB):
            pltpu.async_copy(hs_hbm.at[idx_v[b].at[jj]],
                             rows_v[b].at[pl.ds(jj * IPS, IPS)], sem_g[b])
        pltpu.async_copy(lat_hbm.at[g], lat_v[b], sem_l[b])

    def wait_in(b):
        for jj in range(NSUB):
            pltpu.make_async_copy(hs_hbm.at[idx_v[b].at[jj]],
                                  rows_v[b].at[pl.ds(jj * IPS, IPS)],
                                  sem_g[b]).wait()
        pltpu.make_async_copy(lat_hbm.at[0], lat_v[b], sem_l[b]).wait()

    def issue_out(b, g):
        pltpu.async_copy(out_v[b], aflow_hbm.at[g], sem_o[b])
        pltpu.async_copy(w_v[b], w_hbm.at[g], sem_o[b])

    def wait_out(b):
        pltpu.make_async_copy(out_v[b], aflow_hbm.at[0], sem_o[b]).wait()
        pltpu.make_async_copy(w_v[b], w_hbm.at[0], sem_o[b]).wait()

    def compute(b):
        rows = rows_v[b]
        lat = lat_v[b]
        out = out_v[b]
        wsc = w_v[b]

        def vbody(v, vc):
            latj = [lat[v, pl.ds(16 * j, 16)] for j in range(4)]
            ssvec = jnp.zeros((16,), jnp.float32)
            for k in range(K):
                acc = None
                for j in range(4):
                    d = rows[v * K + k, pl.ds(16 * j, 16)] - latj[j]
                    acc = d * d if acc is None else acc + d * d
                ssvec = jnp.where(kmask[k], jnp.sum(acc), ssvec)
            svec = _newton_sqrt(ssvec)
            dsum = _lane_sum(svec, perms)
            av = alpha_v * dsum
            bv = beta_v / dsum
            wv = (av - jnp.minimum(svec, av)) * bv
            plsc.store_scatter(wsc, [v * K + iota], wv, mask=mask9)
            ojs = list(biasj)
            for k in range(K):
                wk = _permute(wv, kfull[k])
                for j in range(4):
                    ojs[j] = ojs[j] + rows[v * K + k, pl.ds(16 * j, 16)] * wk
            for j in range(4):
                out[v, pl.ds(16 * j, 16)] = ojs[j]
            return vc

        lax.fori_loop(0, C, vbody, 0, unroll=2)

    issue_in(0, wid)

    def outer(t2, carry):
        for b in (0, 1):
            t = t2 * 2 + b
            g = wid + t * NW
            gn = g + NW

            @pl.when(gn < G)
            def _prefetch(b=b, gn=gn):
                issue_in(1 - b, gn)

            @pl.when(g < G)
            def _work(b=b, g=g, t2=t2):
                wait_in(b)

                @pl.when(t2 >= 1)
                def _drain(b=b):
                    wait_out(b)

                compute(b)
                issue_out(b, g)

        return carry

    lax.fori_loop(0, T2MAX, outer, 0)
    wait_out(0)
    wait_out(1)


@jax.jit
def _run(lat3, hidden_state, nbr3, params, bias):
    mesh = plsc.VectorSubcoreMesh(core_axis_name="c", subcore_axis_name="s")
    f = functools.partial(
        pl.kernel,
        mesh=mesh,
        out_type=[
            jax.ShapeDtypeStruct((G, C, F), jnp.float32),
            jax.ShapeDtypeStruct((G, C * K), jnp.float32),
        ],
        scratch_types=[
            pltpu.VMEM((NSUB, IPS), jnp.int32),
            pltpu.VMEM((NSUB, IPS), jnp.int32),
            pltpu.VMEM((C * K, F), jnp.float32),
            pltpu.VMEM((C * K, F), jnp.float32),
            pltpu.VMEM((C, F), jnp.float32),
            pltpu.VMEM((C, F), jnp.float32),
            pltpu.VMEM((C, F), jnp.float32),
            pltpu.VMEM((C, F), jnp.float32),
            pltpu.VMEM((C * K,), jnp.float32),
            pltpu.VMEM((C * K,), jnp.float32),
            pltpu.VMEM((2, 16), jnp.float32),
            pltpu.VMEM((F,), jnp.float32),
            pltpu.SemaphoreType.DMA,
            pltpu.SemaphoreType.DMA,
            pltpu.SemaphoreType.DMA,
            pltpu.SemaphoreType.DMA,
            pltpu.SemaphoreType.DMA,
            pltpu.SemaphoreType.DMA,
        ],
        compiler_params=pltpu.CompilerParams(needs_layout_passes=False,
                                             use_tc_tiling_on_sc=False),
    )(_sc_body)
    return f(lat3, hidden_state, nbr3, params, bias)


def kernel(lattice_values, hidden_state, neighbor_idx, alpha, beta, bias):
    nbr3 = neighbor_idx.reshape(G, NSUB, IPS)
    lat3 = lattice_values.reshape(G, C, F)
    params = jnp.stack([jnp.full((16,), alpha, jnp.float32),
                        jnp.full((16,), beta, jnp.float32)])
    aflow3, w2 = _run(lat3, hidden_state, nbr3, params, bias)
    return aflow3.reshape(N, F), w2.reshape(N, K), neighbor_idx


# speedup vs baseline: 1.0271x; 1.0027x over previous
import functools

import jax
import jax.numpy as jnp
from jax import lax
from jax.experimental import pallas as pl
from jax.experimental.pallas import tpu as pltpu
from jax.experimental.pallas import tpu_sc as plsc

N = 100000
F = 64
K = 9
C = 80
G = N // C
NW = 32
IPS = 120
NSUB = C * K // IPS
CR = C // 2
T2MAX = (G + 2 * NW - 1) // (2 * NW)


def _lane_sum(x, perms):
    for p in perms:
        x = x + lax.gather(x, p, _GDN, (1,),
                           mode=lax.GatherScatterMode.PROMISE_IN_BOUNDS)
    return x


_GDN = lax.GatherDimensionNumbers(offset_dims=(), collapsed_slice_dims=(0,),
                                  start_index_map=(0,))


def _permute(x, idx2d):
    return lax.gather(x, idx2d, _GDN, (1,),
                      mode=lax.GatherScatterMode.PROMISE_IN_BOUNDS)


def _newton_sqrt(x):
    i = lax.bitcast_convert_type(x, jnp.int32)
    y = lax.bitcast_convert_type(jnp.int32(0x5F3759DF) - (i >> 1), jnp.float32)
    hx = 0.5 * x
    for _ in range(2):
        y = y * (1.5 - hx * y * y)
    return x * y


def _sc_body(lat_hbm, hs_hbm, nbr_hbm, par_hbm, bias_hbm,
             aflow_hbm, w_hbm,
             idx0, idx1, rows0, rows1, lat0, lat1, out0, out1, wv0, wv1,
             par_v, bias_v,
             sg0, sg1, sl0, sl1, so0, so1):
    idx_v = (idx0, idx1)
    rows_v = (rows0, rows1)
    lat_v = (lat0, lat1)
    out_v = (out0, out1)
    w_v = (wv0, wv1)
    sem_g = (sg0, sg1)
    sem_l = (sl0, sl1)
    sem_o = (so0, so1)

    wid = lax.axis_index("s") * 2 + lax.axis_index("c")
    pltpu.sync_copy(par_hbm, par_v)
    pltpu.sync_copy(bias_hbm, bias_v)
    alpha_v = par_v[pl.ds(0, 16)]
    beta_v = par_v[pl.ds(16, 16)]
    biasj = [bias_v[pl.ds(16 * j, 16)] for j in range(4)]
    iota = lax.iota(jnp.int32, 16)
    mask9 = iota < K
    perms = [jnp.bitwise_xor(iota, sh)[:, None] for sh in (8, 4, 2, 1)]
    kfull = [jnp.full((16, 1), k, jnp.int32) for k in range(K)]
    kmask = [iota == k for k in range(K)]

    def issue_in(b, g):
        pltpu.sync_copy(nbr_hbm.at[pl.ds(g * C * K, C * K)], idx_v[b])
        for jj in range(NSUB):
            pltpu.async_copy(hs_hbm.at[idx_v[b].at[pl.ds(jj * IPS, IPS)]],
                             rows_v[b].at[pl.ds(jj * IPS, IPS)], sem_g[b])
        pltpu.async_copy(lat_hbm.at[pl.ds(g * CR, CR)], lat_v[b], sem_l[b])

    def wait_in(b):
        for jj in range(NSUB):
            pltpu.make_async_copy(hs_hbm.at[idx_v[b].at[pl.ds(jj * IPS, IPS)]],
                                  rows_v[b].at[pl.ds(jj * IPS, IPS)],
                                  sem_g[b]).wait()
        pltpu.make_async_copy(lat_hbm.at[pl.ds(0, CR)], lat_v[b],
                              sem_l[b]).wait()

    def issue_out(b, g):
        pltpu.async_copy(out_v[b], aflow_hbm.at[pl.ds(g * CR, CR)], sem_o[b])
        pltpu.async_copy(w_v[b], w_hbm.at[pl.ds(g * C * K, C * K)], sem_o[b])

    def wait_out(b):
        pltpu.make_async_copy(out_v[b], aflow_hbm.at[pl.ds(0, CR)],
                              sem_o[b]).wait()
        pltpu.make_async_copy(w_v[b], w_hbm.at[pl.ds(0, C * K)],
                              sem_o[b]).wait()

    def compute(b):
        rows = rows_v[b]
        lat = lat_v[b]
        out = out_v[b]
        wsc = w_v[b]

        def vbody(v, vc):
            vr = v >> 1
            vo = (v & 1) * 64
            latj = [lat[vr, pl.ds(vo + 16 * j, 16)] for j in range(4)]
            ssvec = jnp.zeros((16,), jnp.float32)
            for k in range(K):
                acc = None
                for j in range(4):
                    d = rows[v * K + k, pl.ds(16 * j, 16)] - latj[j]
                    acc = d * d if acc is None else acc + d * d
                ssvec = jnp.where(kmask[k], jnp.sum(acc), ssvec)
            svec = _newton_sqrt(ssvec)
            dsum = _lane_sum(svec, perms)
            av = alpha_v * dsum
            bv = beta_v / dsum
            wv = (av - jnp.minimum(svec, av)) * bv
            plsc.store_scatter(wsc, [v * K + iota], wv, mask=mask9)
            ojs = list(biasj)
            for k in range(K):
                wk = _permute(wv, kfull[k])
                for j in range(4):
                    ojs[j] = ojs[j] + rows[v * K + k, pl.ds(16 * j, 16)] * wk
            for j in range(4):
                out[vr, pl.ds(vo + 16 * j, 16)] = ojs[j]
            return vc

        lax.fori_loop(0, C, vbody, 0, unroll=2)

    issue_in(0, wid)

    def outer(t2, carry):
        for b in (0, 1):
            t = t2 * 2 + b
            g = wid + t * NW
            gn = g + NW

            @pl.when(gn < G)
            def _prefetch(b=b, gn=gn):
                issue_in(1 - b, gn)

            @pl.when(g < G)
            def _work(b=b, g=g, t2=t2):
                wait_in(b)

                @pl.when(t2 >= 1)
                def _drain(b=b):
                    wait_out(b)

                compute(b)
                issue_out(b, g)

        return carry

    lax.fori_loop(0, T2MAX, outer, 0)
    wait_out(0)
    wait_out(1)


@jax.jit
def _run(lattice_values, hidden_state, neighbor_idx, params, bias):
    lat128 = lattice_values.reshape(N // 2, 128)
    nbr1d = neighbor_idx.reshape(N * K)
    mesh = plsc.VectorSubcoreMesh(core_axis_name="c", subcore_axis_name="s")
    f = functools.partial(
        pl.kernel,
        mesh=mesh,
        out_type=[
            jax.ShapeDtypeStruct((N // 2, 128), jnp.float32),
            jax.ShapeDtypeStruct((N * K,), jnp.float32),
        ],
        scratch_types=[
            pltpu.VMEM((C * K,), jnp.int32),
            pltpu.VMEM((C * K,), jnp.int32),
            pltpu.VMEM((C * K, F), jnp.float32),
            pltpu.VMEM((C * K, F), jnp.float32),
            pltpu.VMEM((CR, 128), jnp.float32),
            pltpu.VMEM((CR, 128), jnp.float32),
            pltpu.VMEM((CR, 128), jnp.float32),
            pltpu.VMEM((CR, 128), jnp.float32),
            pltpu.VMEM((C * K,), jnp.float32),
            pltpu.VMEM((C * K,), jnp.float32),
            pltpu.VMEM((32,), jnp.float32),
            pltpu.VMEM((F,), jnp.float32),
            pltpu.SemaphoreType.DMA,
            pltpu.SemaphoreType.DMA,
            pltpu.SemaphoreType.DMA,
            pltpu.SemaphoreType.DMA,
            pltpu.SemaphoreType.DMA,
            pltpu.SemaphoreType.DMA,
        ],
        compiler_params=pltpu.CompilerParams(needs_layout_passes=False,
                                             use_tc_tiling_on_sc=False),
    )(_sc_body)
    aflow2, w1 = f(lat128, hidden_state, nbr1d, params, bias)
    return aflow2.reshape(N, F), w1.reshape(N, K)


def kernel(lattice_values, hidden_state, neighbor_idx, alpha, beta, bias):
    params = jnp.concatenate([jnp.full((16,), alpha, jnp.float32),
                              jnp.full((16,), beta, jnp.float32)])
    aflow, w = _run(lattice_values, hidden_state, neighbor_idx, params, bias)
    return aflow, w, neighbor_idx


# speedup vs baseline: 1.0779x; 1.0495x over previous
import functools

import jax
import jax.numpy as jnp
from jax import lax
from jax.experimental import pallas as pl
from jax.experimental.pallas import tpu as pltpu
from jax.experimental.pallas import tpu_sc as plsc

N = 100000
F = 64
K = 9
C = 80
G = N // C
NW = 32
IPS = 120
NSUB = C * K // IPS
T2MAX = (G + 2 * NW - 1) // (2 * NW)

_GDN = lax.GatherDimensionNumbers(offset_dims=(), collapsed_slice_dims=(0,),
                                  start_index_map=(0,))


def _permute(x, idx2d):
    return lax.gather(x, idx2d, _GDN, (1,),
                      mode=lax.GatherScatterMode.PROMISE_IN_BOUNDS)


def _lane_sum(x, perms):
    for p in perms:
        x = x + _permute(x, p)
    return x


def _newton_sqrt(x):
    i = lax.bitcast_convert_type(x, jnp.int32)
    y = lax.bitcast_convert_type(jnp.int32(0x5F3759DF) - (i >> 1), jnp.float32)
    hx = 0.5 * x
    for _ in range(2):
        y = y * (1.5 - hx * y * y)
    return x * y


def _sc_body(lat_hbm, hs_hbm, nbr_hbm, par_hbm, bias_hbm,
             aflow_hbm, w_hbm,
             idx0, idx1, rows0, rows1, lat0, lat1, out0, out1, wv0, wv1,
             par_v, bias_v,
             sg0, sg1, sl0, sl1, so0, so1):
    idx_v = (idx0, idx1)
    rows_v = (rows0, rows1)
    lat_v = (lat0, lat1)
    out_v = (out0, out1)
    w_v = (wv0, wv1)
    sem_g = (sg0, sg1)
    sem_l = (sl0, sl1)
    sem_o = (so0, so1)

    wid = lax.axis_index("s") * 2 + lax.axis_index("c")
    pltpu.sync_copy(par_hbm, par_v)
    pltpu.sync_copy(bias_hbm, bias_v)
    alpha_v = par_v[pl.ds(0, 16)]
    beta_v = par_v[pl.ds(16, 16)]
    biasj = [bias_v[pl.ds(16 * j, 16)] for j in range(4)]
    iota = lax.iota(jnp.int32, 16)
    mask9 = iota < K
    perms = [jnp.bitwise_xor(iota, sh)[:, None] for sh in (8, 4, 2, 1)]
    kfull = [jnp.full((16, 1), k, jnp.int32) for k in range(K)]
    kmask = [iota == k for k in range(K)]

    def issue_in(b, g):
        pltpu.sync_copy(nbr_hbm.at[g], idx_v[b])
        for jj in range(NSUB):
            pltpu.async_copy(
                hs_hbm.at[idx_v[b].at[jj]],
                rows_v[b].at[pl.ds(jj * IPS, IPS)], sem_g[b])
        pltpu.async_copy(lat_hbm.at[pl.ds(g * C, C)], lat_v[b], sem_l[b])

    def wait_in(b):
        for jj in range(NSUB):
            pltpu.make_async_copy(
                hs_hbm.at[idx_v[b].at[jj]],
                rows_v[b].at[pl.ds(jj * IPS, IPS)], sem_g[b]).wait()
        pltpu.make_async_copy(lat_hbm.at[pl.ds(0, C)], lat_v[b],
                              sem_l[b]).wait()

    def issue_out(b, g):
        pltpu.async_copy(out_v[b], aflow_hbm.at[pl.ds(g * C, C)], sem_o[b])
        pltpu.async_copy(w_v[b], w_hbm.at[pl.ds(g * C, C)], sem_o[b])

    def wait_out(b):
        pltpu.make_async_copy(out_v[b], aflow_hbm.at[pl.ds(0, C)],
                              sem_o[b]).wait()
        pltpu.make_async_copy(w_v[b], w_hbm.at[pl.ds(0, C)], sem_o[b]).wait()

    def compute(b):
        rows = rows_v[b]
        lat = lat_v[b]
        out = out_v[b]
        wsc = w_v[b]

        def vbody(v, vc):
            latj = [lat[v, pl.ds(16 * j, 16)] for j in range(4)]
            ssvec = jnp.zeros((16,), jnp.float32)
            for k in range(K):
                acc = None
                for j in range(4):
                    d = rows[v * K + k, pl.ds(16 * j, 16)] - latj[j]
                    acc = d * d if acc is None else acc + d * d
                ssvec = jnp.where(kmask[k], jnp.sum(acc), ssvec)
            svec = _newton_sqrt(ssvec)
            dsum = _lane_sum(svec, perms)
            av = alpha_v * dsum
            bv = beta_v / dsum
            wv = (av - jnp.minimum(svec, av)) * bv
            vfull = jnp.full((16,), 1, jnp.int32) * v
            plsc.store_scatter(wsc, [vfull, iota], wv, mask=mask9)
            ojs = list(biasj)
            for k in range(K):
                wk = _permute(wv, kfull[k])
                for j in range(4):
                    ojs[j] = ojs[j] + rows[v * K + k, pl.ds(16 * j, 16)] * wk
            for j in range(4):
                out[v, pl.ds(16 * j, 16)] = ojs[j]
            return vc

        lax.fori_loop(0, C, vbody, 0, unroll=2)

    issue_in(0, wid)

    def outer(t2, carry):
        for b in (0, 1):
            t = t2 * 2 + b
            g = wid + t * NW
            gn = g + NW

            @pl.when(gn < G)
            def _prefetch(b=b, gn=gn):
                issue_in(1 - b, gn)

            @pl.when(g < G)
            def _work(b=b, g=g, t2=t2):
                wait_in(b)

                @pl.when(t2 >= 1)
                def _drain(b=b):
                    wait_out(b)

                compute(b)
                issue_out(b, g)

        return carry

    lax.fori_loop(0, T2MAX, outer, 0)
    wait_out(0)
    wait_out(1)


@jax.jit
def _run(lattice_values, hidden_state, neighbor_idx, params, bias):
    nbr3 = neighbor_idx.reshape(G, NSUB, IPS)
    mesh = plsc.VectorSubcoreMesh(core_axis_name="c", subcore_axis_name="s")
    f = functools.partial(
        pl.kernel,
        mesh=mesh,
        out_type=[
            jax.ShapeDtypeStruct((N, F), jnp.float32),
            jax.ShapeDtypeStruct((N, K), jnp.float32),
        ],
        scratch_types=[
            pltpu.VMEM((NSUB, IPS), jnp.int32),
            pltpu.VMEM((NSUB, IPS), jnp.int32),
            pltpu.VMEM((C * K, F), jnp.float32),
            pltpu.VMEM((C * K, F), jnp.float32),
            pltpu.VMEM((C, F), jnp.float32),
            pltpu.VMEM((C, F), jnp.float32),
            pltpu.VMEM((C, F), jnp.float32),
            pltpu.VMEM((C, F), jnp.float32),
            pltpu.VMEM((C, K), jnp.float32),
            pltpu.VMEM((C, K), jnp.float32),
            pltpu.VMEM((32,), jnp.float32),
            pltpu.VMEM((F,), jnp.float32),
            pltpu.SemaphoreType.DMA,
            pltpu.SemaphoreType.DMA,
            pltpu.SemaphoreType.DMA,
            pltpu.SemaphoreType.DMA,
            pltpu.SemaphoreType.DMA,
            pltpu.SemaphoreType.DMA,
        ],
        compiler_params=pltpu.CompilerParams(needs_layout_passes=False,
                                             use_tc_tiling_on_sc=False),
    )(_sc_body)
    return f(lattice_values, hidden_state, nbr3, params, bias)


def kernel(lattice_values, hidden_state, neighbor_idx, alpha, beta, bias):
    params = jnp.concatenate([jnp.full((16,), alpha, jnp.float32),
                              jnp.full((16,), beta, jnp.float32)])
    aflow, w = _run(lattice_values, hidden_state, neighbor_idx, params, bias)
    return aflow, w, neighbor_idx
